# baseline (speedup 1.0000x reference)
import functools

import jax
import jax.numpy as jnp
import numpy as np
from jax.experimental import pallas as pl

_NEG_POS_RATIO = 3
_INT_MIN = np.int32(np.uint32(0x80000000))


def _bit(b):
    return np.int32(np.uint32(1 << b))


def _logsumexp_body(conf_ref, lab_ref, p_ref, nll_ref):
    x = conf_ref[0]
    lab = lab_ref[0]
    P, C = x.shape
    e = jnp.exp(x)
    lse = jnp.log(jnp.sum(e, axis=1, keepdims=True))
    cls = jax.lax.broadcasted_iota(jnp.int32, (P, C), 1)
    x_lab = jnp.sum(jnp.where(cls == lab, x, 0.0), axis=1, keepdims=True)
    x_0 = x[:, 0:1]
    nll_ref[0] = lse - x_lab
    p_ref[0] = lse - x_0


def _mine_body(p_ref, nll_ref, lab_ref, out_ref):
    p = p_ref[...]
    nll = nll_ref[...]
    lab = lab_ref[...]
    N, P = p.shape

    pos = lab > 0
    npos = jnp.sum(pos.astype(jnp.int32), axis=1, keepdims=True)
    negc = P - npos
    keff = jnp.minimum(_NEG_POS_RATIO * npos, negc)

    i = jax.lax.bitcast_convert_type(p, jnp.int32)
    key = jnp.where(i >= 0, i, i ^ np.int32(0x7FFFFFFF))
    key = jnp.where(pos, _INT_MIN, key)
    u = key ^ _INT_MIN

    kp = jnp.maximum(keff, 1)
    prefix = jnp.zeros((N, 1), jnp.int32)
    kr = kp
    for b in range(31, -1, -1):
        cand = prefix | _bit(b)
        match = (jax.lax.shift_right_logical(u, b)
                 == jax.lax.shift_right_logical(cand, b))
        cnt1 = jnp.sum(match.astype(jnp.int32), axis=1, keepdims=True)
        take = cnt1 >= kr
        prefix = jnp.where(take, cand, prefix)
        kr = jnp.where(take, kr, kr - cnt1)
    t_key = prefix ^ _INT_MIN

    gt = key > t_key
    eq = key == t_key
    eqi = eq.astype(jnp.int32)
    cnt_gt = jnp.sum(gt.astype(jnp.int32), axis=1, keepdims=True)
    m = kp - cnt_gt

    idx = jax.lax.broadcasted_iota(jnp.int32, (N, P), 1)
    t = jnp.zeros((N, 1), jnp.int32)
    for b in range(13, -1, -1):
        cand_t = t + _bit(b)
        cnt = jnp.sum(jnp.where(idx < cand_t, eqi, 0), axis=1, keepdims=True)
        t = jnp.where(cnt < m, cand_t, t)
    tie_sel = eq & (idx <= t)

    sel_neg = gt | tie_sel
    neg_num = jnp.sum(jnp.where(sel_neg, nll, 0.0), axis=1, keepdims=True)
    pos_num = jnp.sum(jnp.where(pos, nll, 0.0), axis=1, keepdims=True)
    row_num = pos_num + jnp.where(keff > 0, neg_num, 0.0)
    row_den = (npos + keff).astype(jnp.float32)

    ce = jnp.sum(row_num) / jnp.sum(row_den)
    out_ref[...] = ce / npos.astype(jnp.float32)


_PROBE = 3


def _probe_body(x_ref, o_ref):
    o_ref[0] = jnp.full((1, 128), jnp.sum(x_ref[...]), jnp.float32)


def _dma_probe(confidence):
    N, P, C = confidence.shape
    if _PROBE == 1:
        arr, grid, blk = confidence, (N,), (1, P, C)
    elif _PROBE == 2:
        arr, grid, blk = confidence.reshape(N, P // 4, 4 * C), (N,), (1, P // 4, 4 * C)
    else:
        arr, grid, blk = confidence.reshape(81, 2183, 128), (81,), (1, 2183, 128)
    nb = grid[0]
    ndim = len(blk)
    if ndim == 3:
        ispec = pl.BlockSpec(blk, lambda i: (i, 0, 0))
    else:
        ispec = pl.BlockSpec(blk, lambda i: (i, 0))
        def _probe2(x_ref, o_ref):
            o_ref[0] = jnp.full((1, 128), jnp.sum(x_ref[...]), jnp.float32)
    body = _probe_body
    o = pl.pallas_call(
        body,
        grid=grid,
        in_specs=[ispec],
        out_specs=pl.BlockSpec((1, 1, 128), lambda i: (i, 0, 0)),
        out_shape=jax.ShapeDtypeStruct((nb, 1, 128), jnp.float32),
    )(arr)
    return jnp.sum(o[:, 0, 0]).reshape(1, 1) * jnp.ones((32, 1), jnp.float32)


def _kernel_impl(confidence, pred_loc, oracle_class_labels, oracle_bbox_loc,
                 interpret=False):
    return _dma_probe(confidence)
    del pred_loc, oracle_bbox_loc
    N, P, C = confidence.shape
    lab3 = oracle_class_labels.reshape(N, P, 1)

    p3, nll3 = pl.pallas_call(
        _logsumexp_body,
        grid=(N,),
        in_specs=[
            pl.BlockSpec((1, P, C), lambda i: (i, 0, 0)),
            pl.BlockSpec((1, P, 1), lambda i: (i, 0, 0)),
        ],
        out_specs=[
            pl.BlockSpec((1, P, 1), lambda i: (i, 0, 0)),
            pl.BlockSpec((1, P, 1), lambda i: (i, 0, 0)),
        ],
        out_shape=[
            jax.ShapeDtypeStruct((N, P, 1), jnp.float32),
            jax.ShapeDtypeStruct((N, P, 1), jnp.float32),
        ],
        interpret=interpret,
    )(confidence, lab3)

    if True:
        return (jnp.sum(p3) + jnp.sum(nll3)).reshape(1, 1) * jnp.ones((N, 1), jnp.float32)
    out = pl.pallas_call(
        _mine_body,
        out_shape=jax.ShapeDtypeStruct((N, 1), jnp.float32),
        interpret=interpret,
    )(p3.reshape(N, P), nll3.reshape(N, P), oracle_class_labels)
    return out


def kernel(confidence, pred_loc, oracle_class_labels, oracle_bbox_loc):
    return _kernel_impl(confidence, pred_loc, oracle_class_labels,
                        oracle_bbox_loc)


# speedup vs baseline: 9.4176x; 9.4176x over previous
import functools

import jax
import jax.numpy as jnp
import numpy as np
from jax.experimental import pallas as pl

_NEG_POS_RATIO = 3
_INT_MIN = np.int32(np.uint32(0x80000000))


def _bit(b):
    return np.int32(np.uint32(1 << b))


def _logsumexp_body(conf_ref, lab_ref, p_ref, nll_ref):
    x = conf_ref[0]
    lab = lab_ref[0]
    P, C = x.shape
    dn_minor = (((1,), (1,)), ((), ()))
    dn_std = (((1,), (0,)), ((), ()))
    ident = (jax.lax.broadcasted_iota(jnp.int32, (C, C), 0)
             == jax.lax.broadcasted_iota(jnp.int32, (C, C), 1)
             ).astype(jnp.float32)
    xt = jax.lax.dot_general(ident, x, dn_minor,
                             preferred_element_type=jnp.float32)
    et = jnp.exp(xt)
    ones_row = jnp.ones((1, C), jnp.float32)
    s = jax.lax.dot_general(ones_row, et, dn_std,
                            preferred_element_type=jnp.float32)
    oh = jax.lax.broadcasted_iota(jnp.int32, (C, P), 0) == lab
    xsel = jnp.where(oh, xt, 0.0)
    xl = jax.lax.dot_general(ones_row, xsel, dn_std,
                             preferred_element_type=jnp.float32)
    lse = jnp.log(s)
    nll_ref[0] = lse - xl
    p_ref[0] = lse - xt[0:1, :]


def _mine_body(p_ref, nll_ref, lab_ref, out_ref):
    p = p_ref[...]
    nll = nll_ref[...]
    lab = lab_ref[...]
    N, P = p.shape

    pos = lab > 0
    npos = jnp.sum(pos.astype(jnp.int32), axis=1, keepdims=True)
    negc = P - npos
    keff = jnp.minimum(_NEG_POS_RATIO * npos, negc)
    nposf = npos.astype(jnp.float32)

    def _fast():
        ce = jnp.sum(nll) / jnp.float32(N * P)
        return ce / nposf

    def _slow():
        i = jax.lax.bitcast_convert_type(p, jnp.int32)
        key = jnp.where(i >= 0, i, i ^ np.int32(0x7FFFFFFF))
        key = jnp.where(pos, _INT_MIN, key)
        u = key ^ _INT_MIN

        kp = jnp.maximum(keff, 1)
        prefix = jnp.zeros((N, 1), jnp.int32)
        kr = kp
        for b in range(31, -1, -1):
            cand = prefix | _bit(b)
            match = (jax.lax.shift_right_logical(u, b)
                     == jax.lax.shift_right_logical(cand, b))
            cnt1 = jnp.sum(match.astype(jnp.int32), axis=1, keepdims=True)
            take = cnt1 >= kr
            prefix = jnp.where(take, cand, prefix)
            kr = jnp.where(take, kr, kr - cnt1)
        t_key = prefix ^ _INT_MIN

        gt = key > t_key
        eq = key == t_key
        eqi = eq.astype(jnp.int32)
        cnt_gt = jnp.sum(gt.astype(jnp.int32), axis=1, keepdims=True)
        m = kp - cnt_gt

        idx = jax.lax.broadcasted_iota(jnp.int32, (N, P), 1)
        t = jnp.zeros((N, 1), jnp.int32)
        for b in range(13, -1, -1):
            cand_t = t + _bit(b)
            cnt = jnp.sum(jnp.where(idx < cand_t, eqi, 0),
                          axis=1, keepdims=True)
            t = jnp.where(cnt < m, cand_t, t)
        tie_sel = eq & (idx <= t)

        sel_neg = gt | tie_sel
        neg_num = jnp.sum(jnp.where(sel_neg, nll, 0.0), axis=1, keepdims=True)
        pos_num = jnp.sum(jnp.where(pos, nll, 0.0), axis=1, keepdims=True)
        row_num = pos_num + jnp.where(keff > 0, neg_num, 0.0)
        row_den = (npos + keff).astype(jnp.float32)
        ce = jnp.sum(row_num) / jnp.sum(row_den)
        return ce / nposf

    fast = jnp.all(_NEG_POS_RATIO * npos >= negc)
    out_ref[...] = jax.lax.cond(fast, _fast, _slow)


def _kernel_impl(confidence, pred_loc, oracle_class_labels, oracle_bbox_loc,
                 interpret=False):
    del pred_loc, oracle_bbox_loc
    N, P, C = confidence.shape
    lab3 = oracle_class_labels.reshape(N, 1, P)

    p3, nll3 = pl.pallas_call(
        _logsumexp_body,
        grid=(N,),
        in_specs=[
            pl.BlockSpec((1, P, C), lambda i: (i, 0, 0)),
            pl.BlockSpec((1, 1, P), lambda i: (i, 0, 0)),
        ],
        out_specs=[
            pl.BlockSpec((1, 1, P), lambda i: (i, 0, 0)),
            pl.BlockSpec((1, 1, P), lambda i: (i, 0, 0)),
        ],
        out_shape=[
            jax.ShapeDtypeStruct((N, 1, P), jnp.float32),
            jax.ShapeDtypeStruct((N, 1, P), jnp.float32),
        ],
        interpret=interpret,
    )(confidence, lab3)

    out = pl.pallas_call(
        _mine_body,
        out_shape=jax.ShapeDtypeStruct((N, 1), jnp.float32),
        interpret=interpret,
    )(p3.reshape(N, P), nll3.reshape(N, P), oracle_class_labels)
    return out


def kernel(confidence, pred_loc, oracle_class_labels, oracle_bbox_loc):
    return _kernel_impl(confidence, pred_loc, oracle_class_labels,
                        oracle_bbox_loc)


# speedup vs baseline: 9.8164x; 1.0424x over previous
import functools

import jax
import jax.numpy as jnp
import numpy as np
from jax.experimental import pallas as pl

_NEG_POS_RATIO = 3
_INT_MIN = np.int32(np.uint32(0x80000000))


def _bit(b):
    return np.int32(np.uint32(1 << b))


def _logsumexp_body(conf_ref, lab_ref, p_ref, nll_ref):
    B = conf_ref.shape[0]
    dn_minor = (((1,), (1,)), ((), ()))
    dn_std = (((1,), (0,)), ((), ()))
    for k in range(B):
        x = conf_ref[k]
        lab = lab_ref[k]
        P, C = x.shape
        ident = (jax.lax.broadcasted_iota(jnp.int32, (C, C), 0)
                 == jax.lax.broadcasted_iota(jnp.int32, (C, C), 1)
                 ).astype(jnp.float32)
        xt = jax.lax.dot_general(ident, x, dn_minor,
                                 preferred_element_type=jnp.float32)
        et = jnp.exp(xt)
        ones_row = jnp.ones((1, C), jnp.float32)
        s = jax.lax.dot_general(ones_row, et, dn_std,
                                preferred_element_type=jnp.float32)
        oh = jax.lax.broadcasted_iota(jnp.int32, (C, P), 0) == lab
        xsel = jnp.where(oh, xt, 0.0)
        xl = jax.lax.dot_general(ones_row, xsel, dn_std,
                                 preferred_element_type=jnp.float32)
        lse = jnp.log(s)
        nll_ref[k] = lse - xl
        p_ref[k] = lse - xt[0:1, :]


def _mine_body(p_ref, nll_ref, lab_ref, out_ref):
    p = p_ref[...]
    nll = nll_ref[...]
    lab = lab_ref[...]
    N, P = p.shape

    pos = lab > 0
    npos = jnp.sum(pos.astype(jnp.int32), axis=1, keepdims=True)
    negc = P - npos
    keff = jnp.minimum(_NEG_POS_RATIO * npos, negc)
    nposf = npos.astype(jnp.float32)

    def _fast():
        ce = jnp.sum(nll) / jnp.float32(N * P)
        return ce / nposf

    def _slow():
        i = jax.lax.bitcast_convert_type(p, jnp.int32)
        key = jnp.where(i >= 0, i, i ^ np.int32(0x7FFFFFFF))
        key = jnp.where(pos, _INT_MIN, key)
        u = key ^ _INT_MIN

        kp = jnp.maximum(keff, 1)
        prefix = jnp.zeros((N, 1), jnp.int32)
        kr = kp
        for b in range(31, -1, -1):
            cand = prefix | _bit(b)
            match = (jax.lax.shift_right_logical(u, b)
                     == jax.lax.shift_right_logical(cand, b))
            cnt1 = jnp.sum(match.astype(jnp.int32), axis=1, keepdims=True)
            take = cnt1 >= kr
            prefix = jnp.where(take, cand, prefix)
            kr = jnp.where(take, kr, kr - cnt1)
        t_key = prefix ^ _INT_MIN

        gt = key > t_key
        eq = key == t_key
        eqi = eq.astype(jnp.int32)
        cnt_gt = jnp.sum(gt.astype(jnp.int32), axis=1, keepdims=True)
        m = kp - cnt_gt

        idx = jax.lax.broadcasted_iota(jnp.int32, (N, P), 1)
        t = jnp.zeros((N, 1), jnp.int32)
        for b in range(13, -1, -1):
            cand_t = t + _bit(b)
            cnt = jnp.sum(jnp.where(idx < cand_t, eqi, 0),
                          axis=1, keepdims=True)
            t = jnp.where(cnt < m, cand_t, t)
        tie_sel = eq & (idx <= t)

        sel_neg = gt | tie_sel
        neg_num = jnp.sum(jnp.where(sel_neg, nll, 0.0), axis=1, keepdims=True)
        pos_num = jnp.sum(jnp.where(pos, nll, 0.0), axis=1, keepdims=True)
        row_num = pos_num + jnp.where(keff > 0, neg_num, 0.0)
        row_den = (npos + keff).astype(jnp.float32)
        ce = jnp.sum(row_num) / jnp.sum(row_den)
        return ce / nposf

    fast = jnp.all(_NEG_POS_RATIO * npos >= negc)
    out_ref[...] = jax.lax.cond(fast, _fast, _slow)


def _kernel_impl(confidence, pred_loc, oracle_class_labels, oracle_bbox_loc,
                 interpret=False):
    del pred_loc, oracle_bbox_loc
    N, P, C = confidence.shape
    lab3 = oracle_class_labels.reshape(N, 1, P)

    BI = 2
    p3, nll3 = pl.pallas_call(
        _logsumexp_body,
        grid=(N // BI,),
        in_specs=[
            pl.BlockSpec((BI, P, C), lambda i: (i, 0, 0)),
            pl.BlockSpec((BI, 1, P), lambda i: (i, 0, 0)),
        ],
        out_specs=[
            pl.BlockSpec((BI, 1, P), lambda i: (i, 0, 0)),
            pl.BlockSpec((BI, 1, P), lambda i: (i, 0, 0)),
        ],
        out_shape=[
            jax.ShapeDtypeStruct((N, 1, P), jnp.float32),
            jax.ShapeDtypeStruct((N, 1, P), jnp.float32),
        ],
        interpret=interpret,
    )(confidence, lab3)

    out = pl.pallas_call(
        _mine_body,
        out_shape=jax.ShapeDtypeStruct((N, 1), jnp.float32),
        interpret=interpret,
    )(p3.reshape(N, P), nll3.reshape(N, P), oracle_class_labels)
    return out


def kernel(confidence, pred_loc, oracle_class_labels, oracle_bbox_loc):
    return _kernel_impl(confidence, pred_loc, oracle_class_labels,
                        oracle_bbox_loc)


# speedup vs baseline: 9.9592x; 1.0145x over previous
import functools

import jax
import jax.numpy as jnp
import numpy as np
from jax.experimental import pallas as pl

_NEG_POS_RATIO = 3
_INT_MIN = np.int32(np.uint32(0x80000000))


def _bit(b):
    return np.int32(np.uint32(1 << b))


def _logsumexp_body(conf_ref, lab_ref, p_ref, nll_ref):
    B = conf_ref.shape[0]
    dn_minor = (((1,), (1,)), ((), ()))
    dn_std = (((1,), (0,)), ((), ()))
    for k in range(B):
        x = conf_ref[k]
        lab = lab_ref[k]
        P, C = x.shape
        ident = (jax.lax.broadcasted_iota(jnp.int32, (C, C), 0)
                 == jax.lax.broadcasted_iota(jnp.int32, (C, C), 1)
                 ).astype(jnp.float32)
        xt = jax.lax.dot_general(ident, x, dn_minor,
                                 preferred_element_type=jnp.float32)
        et = jnp.exp(xt)
        ones_row = jnp.ones((1, C), jnp.float32)
        s = jax.lax.dot_general(ones_row, et, dn_std,
                                preferred_element_type=jnp.float32)
        oh = jax.lax.broadcasted_iota(jnp.int32, (C, P), 0) == lab
        xsel = jnp.where(oh, xt, 0.0)
        xl = jax.lax.dot_general(ones_row, xsel, dn_std,
                                 preferred_element_type=jnp.float32)
        lse = jnp.log(s)
        nll_ref[k] = lse - xl
        p_ref[k] = lse - xt[0:1, :]


def _mine_body(p_ref, nll_ref, lab_ref, out_ref):
    p = p_ref[...]
    nll = nll_ref[...]
    lab = lab_ref[...]
    N, P = p.shape

    pos = lab > 0
    npos = jnp.sum(pos.astype(jnp.int32), axis=1, keepdims=True)
    negc = P - npos
    keff = jnp.minimum(_NEG_POS_RATIO * npos, negc)
    nposf = npos.astype(jnp.float32)

    def _fast():
        ce = jnp.sum(nll) / jnp.float32(N * P)
        return ce / nposf

    def _slow():
        i = jax.lax.bitcast_convert_type(p, jnp.int32)
        key = jnp.where(i >= 0, i, i ^ np.int32(0x7FFFFFFF))
        key = jnp.where(pos, _INT_MIN, key)
        u = key ^ _INT_MIN

        kp = jnp.maximum(keff, 1)
        prefix = jnp.zeros((N, 1), jnp.int32)
        kr = kp
        for b in range(31, -1, -1):
            cand = prefix | _bit(b)
            match = (jax.lax.shift_right_logical(u, b)
                     == jax.lax.shift_right_logical(cand, b))
            cnt1 = jnp.sum(match.astype(jnp.int32), axis=1, keepdims=True)
            take = cnt1 >= kr
            prefix = jnp.where(take, cand, prefix)
            kr = jnp.where(take, kr, kr - cnt1)
        t_key = prefix ^ _INT_MIN

        gt = key > t_key
        eq = key == t_key
        eqi = eq.astype(jnp.int32)
        cnt_gt = jnp.sum(gt.astype(jnp.int32), axis=1, keepdims=True)
        m = kp - cnt_gt

        idx = jax.lax.broadcasted_iota(jnp.int32, (N, P), 1)
        t = jnp.zeros((N, 1), jnp.int32)
        for b in range(13, -1, -1):
            cand_t = t + _bit(b)
            cnt = jnp.sum(jnp.where(idx < cand_t, eqi, 0),
                          axis=1, keepdims=True)
            t = jnp.where(cnt < m, cand_t, t)
        tie_sel = eq & (idx <= t)

        sel_neg = gt | tie_sel
        neg_num = jnp.sum(jnp.where(sel_neg, nll, 0.0), axis=1, keepdims=True)
        pos_num = jnp.sum(jnp.where(pos, nll, 0.0), axis=1, keepdims=True)
        row_num = pos_num + jnp.where(keff > 0, neg_num, 0.0)
        row_den = (npos + keff).astype(jnp.float32)
        ce = jnp.sum(row_num) / jnp.sum(row_den)
        return ce / nposf

    fast = jnp.all(_NEG_POS_RATIO * npos >= negc)
    out_ref[...] = jax.lax.cond(fast, _fast, _slow)


def _kernel_impl(confidence, pred_loc, oracle_class_labels, oracle_bbox_loc,
                 interpret=False):
    del pred_loc, oracle_bbox_loc
    N, P, C = confidence.shape
    lab3 = oracle_class_labels.reshape(N, 1, P)

    BI = 4
    p3, nll3 = pl.pallas_call(
        _logsumexp_body,
        grid=(N // BI,),
        in_specs=[
            pl.BlockSpec((BI, P, C), lambda i: (i, 0, 0)),
            pl.BlockSpec((BI, 1, P), lambda i: (i, 0, 0)),
        ],
        out_specs=[
            pl.BlockSpec((BI, 1, P), lambda i: (i, 0, 0)),
            pl.BlockSpec((BI, 1, P), lambda i: (i, 0, 0)),
        ],
        out_shape=[
            jax.ShapeDtypeStruct((N, 1, P), jnp.float32),
            jax.ShapeDtypeStruct((N, 1, P), jnp.float32),
        ],
        interpret=interpret,
    )(confidence, lab3)

    out = pl.pallas_call(
        _mine_body,
        out_shape=jax.ShapeDtypeStruct((N, 1), jnp.float32),
        interpret=interpret,
    )(p3.reshape(N, P), nll3.reshape(N, P), oracle_class_labels)
    return out


def kernel(confidence, pred_loc, oracle_class_labels, oracle_bbox_loc):
    return _kernel_impl(confidence, pred_loc, oracle_class_labels,
                        oracle_bbox_loc)
